# Initial kernel scaffold; baseline (speedup 1.0000x reference)
import jax
import jax.numpy as jnp
from jax.experimental import pallas as pl


def kernel(x, edge_index, Wl1, Wr1, b1, Wl2, Wr2, b2):
    raise NotImplementedError("write your pallas kernel here")


# speedup vs baseline: 7.4587x; 7.4587x over previous
import functools

import jax
import jax.numpy as jnp
from jax import lax
from jax.experimental import pallas as pl
from jax.experimental.pallas import tpu as pltpu
from jax.experimental.pallas import tpu_sc as plsc

N = 10000
E = 320000
D = 128

NC = 2
NS = 16
NW = NC * NS
EW = E // NW
KB = 80
KN = EW // KB
NP = 10240
RS = NP // NS

_mesh = plsc.VectorSubcoreMesh(core_axis_name="c", subcore_axis_name="s")


@functools.partial(
    pl.kernel,
    out_type=(
        jax.ShapeDtypeStruct((NC, NP, D), jnp.float32),
        jax.ShapeDtypeStruct((NC, NP), jnp.float32),
    ),
    mesh=_mesh,
    scratch_types=[
        pltpu.VMEM((KN, KB), jnp.int32),
        pltpu.VMEM((KN, KB), jnp.int32),
        pltpu.VMEM((KB, D), jnp.float32),
        pltpu.VMEM((KB,), jnp.float32),
        pltpu.VMEM_SHARED((NP, D), jnp.float32),
        pltpu.VMEM_SHARED((NP,), jnp.float32),
        pltpu.SemaphoreType.DMA,
    ],
)
def _sc_segsum(y_hbm, src_hbm, dst_hbm, znd_hbm, zn_hbm,
               acc_out, cnt_out,
               src_v, dst_v, rows_v, ones_v, acc_sh, cnt_sh, sem):
    cid = lax.axis_index("c")
    sid = lax.axis_index("s")
    wid = sid * NC + cid

    pltpu.sync_copy(src_hbm.at[wid], src_v)
    pltpu.sync_copy(dst_hbm.at[wid], dst_v)
    for i in range(KB // 16):
        ones_v[pl.ds(i * 16, 16)] = jnp.full((16,), 1.0, jnp.float32)

    pltpu.sync_copy(znd_hbm.at[pl.ds(sid * RS, RS)],
                    acc_sh.at[pl.ds(sid * RS, RS)])

    @pl.when(sid == 0)
    def _():
        pltpu.sync_copy(zn_hbm, cnt_sh)

    plsc.subcore_barrier()

    def body(j, carry):
        pltpu.async_copy(y_hbm.at[src_v.at[j]], rows_v, sem).wait()
        pltpu.sync_copy(rows_v, acc_sh.at[dst_v.at[j]], add=True)
        pltpu.sync_copy(ones_v, cnt_sh.at[dst_v.at[j]], add=True)
        return carry

    lax.fori_loop(0, KN, body, 0)
    plsc.subcore_barrier()

    pltpu.sync_copy(acc_sh.at[pl.ds(sid * RS, RS)],
                    acc_out.at[cid, pl.ds(sid * RS, RS)])

    @pl.when(sid == 0)
    def _():
        pltpu.sync_copy(cnt_sh, cnt_out.at[cid])


BR = 1000


def _pre_body(x_ref, wl_ref, wr_ref, b_ref, y_ref, r_ref):
    xb = x_ref[...]
    y_ref[...] = jnp.dot(xb, wl_ref[...], preferred_element_type=jnp.float32)
    r_ref[...] = (jnp.dot(xb, wr_ref[...], preferred_element_type=jnp.float32)
                  + b_ref[...])


_pre = pl.pallas_call(
    _pre_body,
    grid=(N // BR,),
    in_specs=[
        pl.BlockSpec((BR, D), lambda i: (i, 0)),
        pl.BlockSpec((D, D), lambda i: (0, 0)),
        pl.BlockSpec((D, D), lambda i: (0, 0)),
        pl.BlockSpec((1, D), lambda i: (0, 0)),
    ],
    out_specs=[pl.BlockSpec((BR, D), lambda i: (i, 0)),
               pl.BlockSpec((BR, D), lambda i: (i, 0))],
    out_shape=[jax.ShapeDtypeStruct((N, D), jnp.float32)] * 2,
)


def _mid_body(agg_ref, cnt_ref, r_ref, wl_ref, wr_ref, b_ref, y_ref, r2_ref):
    agg = agg_ref[0] + agg_ref[1]
    cnt = cnt_ref[...]
    inv = 1.0 / jnp.maximum(cnt[:, 0:1] + cnt[:, 1:2], 1.0)
    h = jnp.maximum(agg * inv + r_ref[...], 0.0)
    y_ref[...] = jnp.dot(h, wl_ref[...], preferred_element_type=jnp.float32)
    r2_ref[...] = (jnp.dot(h, wr_ref[...], preferred_element_type=jnp.float32)
                   + b_ref[...])


_mid = pl.pallas_call(
    _mid_body,
    grid=(N // BR,),
    in_specs=[
        pl.BlockSpec((NC, BR, D), lambda i: (0, i, 0)),
        pl.BlockSpec((BR, NC), lambda i: (i, 0)),
        pl.BlockSpec((BR, D), lambda i: (i, 0)),
        pl.BlockSpec((D, D), lambda i: (0, 0)),
        pl.BlockSpec((D, D), lambda i: (0, 0)),
        pl.BlockSpec((1, D), lambda i: (0, 0)),
    ],
    out_specs=[pl.BlockSpec((BR, D), lambda i: (i, 0)),
               pl.BlockSpec((BR, D), lambda i: (i, 0))],
    out_shape=[jax.ShapeDtypeStruct((N, D), jnp.float32)] * 2,
)


def _post_body(agg_ref, cnt_ref, r_ref, o_ref):
    agg = agg_ref[0] + agg_ref[1]
    cnt = cnt_ref[...]
    inv = 1.0 / jnp.maximum(cnt[:, 0:1] + cnt[:, 1:2], 1.0)
    o_ref[...] = jnp.maximum(agg * inv + r_ref[...], 0.0)


_post = pl.pallas_call(
    _post_body,
    grid=(N // BR,),
    in_specs=[
        pl.BlockSpec((NC, BR, D), lambda i: (0, i, 0)),
        pl.BlockSpec((BR, NC), lambda i: (i, 0)),
        pl.BlockSpec((BR, D), lambda i: (i, 0)),
    ],
    out_specs=pl.BlockSpec((BR, D), lambda i: (i, 0)),
    out_shape=jax.ShapeDtypeStruct((N, D), jnp.float32),
)


def kernel(x, edge_index, Wl1, Wr1, b1, Wl2, Wr2, b2):
    ei = edge_index.astype(jnp.int32)
    src = ei[0].reshape(NW, KN, KB)
    dst = ei[1].reshape(NW, KN, KB)
    znd = jnp.zeros((NP, D), jnp.float32)
    zn = jnp.zeros((NP,), jnp.float32)

    y1, r1 = _pre(x, Wl1, Wr1, b1.reshape(1, D))
    agg1, cnt1 = _sc_segsum(y1, src, dst, znd, zn)
    cnt_t = cnt1.T
    y2, r2 = _mid(agg1, cnt_t, r1, Wl2, Wr2, b2.reshape(1, D))
    agg2, _ = _sc_segsum(y2, src, dst, znd, zn)
    return _post(agg2, cnt_t, r2)
